# baseline (speedup 1.0000x reference)
import jax
import jax.numpy as jnp
from jax import lax
from jax.experimental import pallas as pl
from jax.experimental.pallas import tpu as pltpu
from jax.experimental.pallas import tpu_sc as plsc

_N = 10000
_THR = 0.01

_EP = 323584
_BT = _EP // 16
_HT = _BT // 2
_QT = _EP // 32
_GARB = 2 ** 18
_TBL = 2 ** 18 + 16

_HASH_LEVELS = (
    (2654435761, 14),
    (2246822519, 14),
    (3266489917, 14),
    (668265263, 14),
    (374761393, 14),
    (4252082373, 14),
)


def _join_body(src_h, dst_h, w_h, out_h, kbuf, wbuf, qbuf, acc, idx1,
               win, vst, neg1, zf, tkeys, tvals):
    s = lax.axis_index("s")
    c = lax.axis_index("c")
    wid = s * 2 + c
    bbase = s * _BT
    qbase = wid * _QT

    pltpu.sync_copy(src_h.at[pl.ds(bbase, _BT)], kbuf)
    pltpu.sync_copy(w_h.at[pl.ds(bbase, _BT)], wbuf)
    for half in (0, 1):
        pltpu.sync_copy(dst_h.at[pl.ds(bbase + half * _HT, _HT)], win)

        def mk_keys(g, carry):
            sl = pl.ds(half * _HT + g * 16, 16)
            kbuf[sl] = kbuf[sl] * _N + win[pl.ds(g * 16, 16)]
            return carry

        lax.fori_loop(0, _HT // 16, mk_keys, 0)

    pltpu.sync_copy(src_h.at[pl.ds(qbase, _QT)], qbuf)
    pltpu.sync_copy(dst_h.at[pl.ds(qbase, _QT)], win)

    def mk_queries(g, carry):
        sl = pl.ds(g * 16, 16)
        s16 = qbuf[sl]
        d16 = win[sl]
        qbuf[sl] = jnp.minimum(s16, d16) * _N + jnp.maximum(s16, d16)
        acc[sl] = jnp.zeros((16,), jnp.float32)
        return carry

    lax.fori_loop(0, _QT // 16, mk_queries, 0)

    def init_consts(i, carry):
        neg1[pl.ds(i * 16, 16)] = jnp.full((16,), -1, jnp.int32)
        zf[pl.ds(i * 16, 16)] = jnp.zeros((16,), jnp.float32)
        return carry

    lax.fori_loop(0, 128, init_consts, 0)

    stripe = (2 ** 18) // 16
    sbase = s * stripe

    for cst, sh in _HASH_LEVELS:
        ucst = jnp.uint32(cst)
        ush = jnp.uint32(sh)

        for off in range(0, stripe, 2048):
            pltpu.sync_copy(neg1, tkeys.at[pl.ds(sbase + off, 2048)])
            pltpu.sync_copy(zf, tvals.at[pl.ds(sbase + off, 2048)])
        plsc.subcore_barrier()

        def hash_strip(buf, base):
            def hgrp(g, carry):
                k16 = buf[pl.ds(base + g * 16, 16)]
                h = (k16.astype(jnp.uint32) * ucst) >> ush
                idx1[pl.ds(g * 16, 16)] = jnp.where(
                    k16 < 0, jnp.int32(_GARB), h.astype(jnp.int32))
                return carry

            lax.fori_loop(0, _HT // 16, hgrp, 0)

        for half in (0, 1):
            hash_strip(kbuf, half * _HT)
            pltpu.sync_copy(kbuf.at[pl.ds(half * _HT, _HT)],
                            tkeys.at[idx1])
        plsc.subcore_barrier()

        for half in (0, 1):
            hash_strip(kbuf, half * _HT)
            pltpu.sync_copy(tkeys.at[idx1], win)

            def bfin(g, carry):
                sl = pl.ds(half * _HT + g * 16, 16)
                j16 = pl.ds(g * 16, 16)
                k16 = kbuf[sl]
                won = (win[j16] == k16) & (k16 >= 0)
                vst[j16] = jnp.where(won, wbuf[sl], 0.0)
                kbuf[sl] = jnp.where(won, -5, k16)
                return carry

            lax.fori_loop(0, _HT // 16, bfin, 0)
            pltpu.sync_copy(vst, tvals.at[idx1], add=True)
        plsc.subcore_barrier()

        hash_strip(qbuf, 0)
        pltpu.sync_copy(tkeys.at[idx1], win)
        pltpu.sync_copy(tvals.at[idx1], vst)

        def qaccum(g, carry):
            sl = pl.ds(g * 16, 16)
            found = win[sl] == qbuf[sl]
            acc[sl] = acc[sl] + jnp.where(found, vst[sl], 0.0)
            return carry

        lax.fori_loop(0, _QT // 16, qaccum, 0)
        plsc.subcore_barrier()

    pltpu.sync_copy(acc, out_h.at[pl.ds(qbase, _QT)])


@jax.jit
def _sc_join(src_p, dst_p, w_p):
    mesh = plsc.VectorSubcoreMesh(core_axis_name="c", subcore_axis_name="s")
    return pl.kernel(
        _join_body,
        out_type=jax.ShapeDtypeStruct((_EP,), jnp.float32),
        mesh=mesh,
        scratch_types=[
            pltpu.VMEM((_BT,), jnp.int32),
            pltpu.VMEM((_BT,), jnp.float32),
            pltpu.VMEM((_QT,), jnp.int32),
            pltpu.VMEM((_QT,), jnp.float32),
            pltpu.VMEM((_HT,), jnp.int32),
            pltpu.VMEM((_HT,), jnp.int32),
            pltpu.VMEM((_HT,), jnp.float32),
            pltpu.VMEM((2048,), jnp.int32),
            pltpu.VMEM((2048,), jnp.float32),
            pltpu.VMEM_SHARED((_TBL,), jnp.int32),
            pltpu.VMEM_SHARED((_TBL,), jnp.float32),
        ],
    )(src_p, dst_p, w_p)


def _epilogue_body(src_ref, dst_ref, inner_ref, aa_ref, bb_ref, wsum_ref,
                   out_ref):
    src = src_ref[...]
    dst = dst_ref[...]
    inner = inner_ref[...]
    denom = jnp.sqrt(aa_ref[...]) * jnp.sqrt(bb_ref[...]) + 1e-08
    keep = (inner / denom) >= _THR
    v = jnp.where(keep, wsum_ref[...], 0.0)
    out_ref[...] = jnp.where(src == dst, 2.0 * v, v)


def kernel(edge_index, edge_weight, features):
    E = edge_index.shape[1]
    src = edge_index[0]
    dst = edge_index[1]

    pad = _EP - E
    src_p = jnp.concatenate([src, jnp.full((pad,), _N, jnp.int32)])
    dst_p = jnp.concatenate([dst, jnp.zeros((pad,), jnp.int32)])
    w_p = jnp.concatenate([edge_weight, jnp.zeros((pad,), jnp.float32)])
    wsum = _sc_join(src_p, dst_p, w_p)[:E]

    fs = features[src]
    fd = features[dst]
    inner = jnp.sum(fs * fd, axis=-1)
    aa = jnp.sum(fs * fs, axis=-1)
    bb = jnp.sum(fd * fd, axis=-1)

    shape2d = (E // 128, 128)
    out = pl.pallas_call(
        _epilogue_body,
        out_shape=jax.ShapeDtypeStruct(shape2d, jnp.float32),
    )(
        src.reshape(shape2d),
        dst.reshape(shape2d),
        inner.reshape(shape2d),
        aa.reshape(shape2d),
        bb.reshape(shape2d),
        wsum.reshape(shape2d),
    )
    return out.reshape(E)


# speedup vs baseline: 1.0111x; 1.0111x over previous
import jax
import jax.numpy as jnp
from jax import lax
from jax.experimental import pallas as pl
from jax.experimental.pallas import tpu as pltpu
from jax.experimental.pallas import tpu_sc as plsc

_N = 10000
_THR = 0.01

_EP = 323584
_BT = _EP // 16
_HT = _BT // 2
_QT = _EP // 32
_GARB = 2 ** 18
_TBL = 2 ** 18 + 16

_HASH_LEVELS = (
    (2654435761, 14),
    (2246822519, 14),
    (3266489917, 14),
    (668265263, 14),
    (374761393, 14),
    (4252082373, 14),
)


def _join_body(src_h, dst_h, w_h, out_h, kbuf, wbuf, qbuf, acc, idx1,
               win, vst, neg1, zf, tkeys, tvals):
    s = lax.axis_index("s")
    c = lax.axis_index("c")
    wid = s * 2 + c
    bbase = s * _BT
    qbase = wid * _QT

    pltpu.sync_copy(src_h.at[pl.ds(bbase, _BT)], kbuf)
    pltpu.sync_copy(w_h.at[pl.ds(bbase, _BT)], wbuf)
    for half in (0, 1):
        pltpu.sync_copy(dst_h.at[pl.ds(bbase + half * _HT, _HT)], win)

        def mk_keys(g, carry):
            for j in range(8):
                sl = pl.ds(half * _HT + g * 128 + j * 16, 16)
                kbuf[sl] = kbuf[sl] * _N + win[pl.ds(g * 128 + j * 16, 16)]
            return carry

        lax.fori_loop(0, _HT // 128, mk_keys, 0)

    pltpu.sync_copy(src_h.at[pl.ds(qbase, _QT)], qbuf)
    pltpu.sync_copy(dst_h.at[pl.ds(qbase, _QT)], win)

    def mk_queries(g, carry):
        for j in range(8):
            sl = pl.ds(g * 128 + j * 16, 16)
            s16 = qbuf[sl]
            d16 = win[sl]
            qbuf[sl] = jnp.minimum(s16, d16) * _N + jnp.maximum(s16, d16)
            acc[sl] = jnp.zeros((16,), jnp.float32)
        return carry

    lax.fori_loop(0, _QT // 128, mk_queries, 0)

    def init_consts(i, carry):
        neg1[pl.ds(i * 16, 16)] = jnp.full((16,), -1, jnp.int32)
        zf[pl.ds(i * 16, 16)] = jnp.zeros((16,), jnp.float32)
        return carry

    lax.fori_loop(0, 128, init_consts, 0)

    stripe = (2 ** 18) // 16
    sbase = s * stripe

    for cst, sh in _HASH_LEVELS:
        ucst = jnp.uint32(cst)
        ush = jnp.uint32(sh)

        for off in range(0, stripe, 2048):
            pltpu.sync_copy(neg1, tkeys.at[pl.ds(sbase + off, 2048)])
            pltpu.sync_copy(zf, tvals.at[pl.ds(sbase + off, 2048)])
        plsc.subcore_barrier()

        def hash_strip(buf, base):
            def hgrp(g, carry):
                for j in range(8):
                    k16 = buf[pl.ds(base + g * 128 + j * 16, 16)]
                    h = (k16.astype(jnp.uint32) * ucst) >> ush
                    idx1[pl.ds(g * 128 + j * 16, 16)] = jnp.where(
                        k16 < 0, jnp.int32(_GARB), h.astype(jnp.int32))
                return carry

            lax.fori_loop(0, _HT // 128, hgrp, 0)

        for half in (0, 1):
            hash_strip(kbuf, half * _HT)
            pltpu.sync_copy(kbuf.at[pl.ds(half * _HT, _HT)],
                            tkeys.at[idx1])
        plsc.subcore_barrier()

        for half in (0, 1):
            hash_strip(kbuf, half * _HT)
            pltpu.sync_copy(tkeys.at[idx1], win)

            def bfin(g, carry):
                for j in range(8):
                    sl = pl.ds(half * _HT + g * 128 + j * 16, 16)
                    j16 = pl.ds(g * 128 + j * 16, 16)
                    k16 = kbuf[sl]
                    won = (win[j16] == k16) & (k16 >= 0)
                    vst[j16] = jnp.where(won, wbuf[sl], 0.0)
                    kbuf[sl] = jnp.where(won, -5, k16)
                return carry

            lax.fori_loop(0, _HT // 128, bfin, 0)
            pltpu.sync_copy(vst, tvals.at[idx1], add=True)
        plsc.subcore_barrier()

        hash_strip(qbuf, 0)
        pltpu.sync_copy(tkeys.at[idx1], win)
        pltpu.sync_copy(tvals.at[idx1], vst)

        def qaccum(g, carry):
            for j in range(8):
                sl = pl.ds(g * 128 + j * 16, 16)
                found = win[sl] == qbuf[sl]
                acc[sl] = acc[sl] + jnp.where(found, vst[sl], 0.0)
            return carry

        lax.fori_loop(0, _QT // 128, qaccum, 0)
        plsc.subcore_barrier()

    pltpu.sync_copy(acc, out_h.at[pl.ds(qbase, _QT)])


@jax.jit
def _sc_join(src_p, dst_p, w_p):
    mesh = plsc.VectorSubcoreMesh(core_axis_name="c", subcore_axis_name="s")
    return pl.kernel(
        _join_body,
        out_type=jax.ShapeDtypeStruct((_EP,), jnp.float32),
        mesh=mesh,
        scratch_types=[
            pltpu.VMEM((_BT,), jnp.int32),
            pltpu.VMEM((_BT,), jnp.float32),
            pltpu.VMEM((_QT,), jnp.int32),
            pltpu.VMEM((_QT,), jnp.float32),
            pltpu.VMEM((_HT,), jnp.int32),
            pltpu.VMEM((_HT,), jnp.int32),
            pltpu.VMEM((_HT,), jnp.float32),
            pltpu.VMEM((2048,), jnp.int32),
            pltpu.VMEM((2048,), jnp.float32),
            pltpu.VMEM_SHARED((_TBL,), jnp.int32),
            pltpu.VMEM_SHARED((_TBL,), jnp.float32),
        ],
    )(src_p, dst_p, w_p)


def _epilogue_body(src_ref, dst_ref, inner_ref, aa_ref, bb_ref, wsum_ref,
                   out_ref):
    src = src_ref[...]
    dst = dst_ref[...]
    inner = inner_ref[...]
    denom = jnp.sqrt(aa_ref[...]) * jnp.sqrt(bb_ref[...]) + 1e-08
    keep = (inner / denom) >= _THR
    v = jnp.where(keep, wsum_ref[...], 0.0)
    out_ref[...] = jnp.where(src == dst, 2.0 * v, v)


def kernel(edge_index, edge_weight, features):
    E = edge_index.shape[1]
    src = edge_index[0]
    dst = edge_index[1]

    pad = _EP - E
    src_p = jnp.concatenate([src, jnp.full((pad,), _N, jnp.int32)])
    dst_p = jnp.concatenate([dst, jnp.zeros((pad,), jnp.int32)])
    w_p = jnp.concatenate([edge_weight, jnp.zeros((pad,), jnp.float32)])
    wsum = _sc_join(src_p, dst_p, w_p)[:E]

    fs = features[src]
    fd = features[dst]
    inner = jnp.sum(fs * fd, axis=-1)
    aa = jnp.sum(fs * fs, axis=-1)
    bb = jnp.sum(fd * fd, axis=-1)

    shape2d = (E // 128, 128)
    out = pl.pallas_call(
        _epilogue_body,
        out_shape=jax.ShapeDtypeStruct(shape2d, jnp.float32),
    )(
        src.reshape(shape2d),
        dst.reshape(shape2d),
        inner.reshape(shape2d),
        aa.reshape(shape2d),
        bb.reshape(shape2d),
        wsum.reshape(shape2d),
    )
    return out.reshape(E)


# speedup vs baseline: 1.4059x; 1.3905x over previous
import functools

import jax
import jax.numpy as jnp
from jax import lax
from jax.experimental import pallas as pl
from jax.experimental.pallas import tpu as pltpu
from jax.experimental.pallas import tpu_sc as plsc

_N = 10000
_THR = 0.01

_EP = 323584
_BT = _EP // 16
_BC = _BT // 128
_QT = _EP // 32
_QC = _QT // 128
_GARB = 2 ** 19
_TBL = 2 ** 19 + 16

_HASH_LEVELS = (
    (2654435761, 13, 2 ** 19),
    (2246822519, 13, 2 ** 19),
    (3266489917, 15, 2 ** 17),
    (668265263, 17, 2 ** 15),
    (374761393, 17, 2 ** 15),
)


def _join_body(src_h, dst_h, w_h, out_h, kbuf, wbuf, qbuf, acc, idxrow,
               win, vst, qk, qv, dstage, neg1, zf, tkeys, tvals,
               saa, sab, sga, sgb):
    s = lax.axis_index("s")
    c = lax.axis_index("c")
    wid = s * 2 + c
    bbase = s * _BT
    qbase = wid * _QT

    pltpu.sync_copy(src_h.at[pl.ds(bbase, _BT)], kbuf)
    pltpu.sync_copy(w_h.at[pl.ds(bbase, _BT)], wbuf)

    def mk_keys(i, carry):
        pltpu.sync_copy(dst_h.at[pl.ds(bbase + i * 256, 256)], dstage)
        for j in range(16):
            sl = pl.ds(i * 256 + j * 16, 16)
            kbuf[sl] = kbuf[sl] * _N + dstage[pl.ds(j * 16, 16)]
        return carry

    lax.fori_loop(0, _BT // 256, mk_keys, 0)

    def init_acc(i, carry):
        acc[pl.ds(i * 16, 16)] = jnp.zeros((16,), jnp.float32)
        return carry

    lax.fori_loop(0, _QT // 16, init_acc, 0)

    def init_consts(i, carry):
        neg1[pl.ds(i * 16, 16)] = jnp.full((16,), -1, jnp.int32)
        zf[pl.ds(i * 16, 16)] = jnp.zeros((16,), jnp.float32)
        return carry

    lax.fori_loop(0, 64, init_consts, 0)

    pltpu.sync_copy(src_h.at[pl.ds(qbase, _QT)], qbuf)

    def mk_queries(i, carry):
        pltpu.sync_copy(dst_h.at[pl.ds(qbase + i * 128, 128)],
                        dstage.at[pl.ds(0, 128)])
        for j in range(8):
            sl = pl.ds(i * 128 + j * 16, 16)
            s16 = qbuf[sl]
            d16 = dstage[pl.ds(j * 16, 16)]
            qbuf[sl] = (jnp.minimum(s16, d16) * _N + jnp.maximum(s16, d16))
        return carry

    lax.fori_loop(0, _QC, mk_queries, 0)

    def wait512(sem):
        pltpu.make_async_copy(src_h.at[pl.ds(0, 128)], qk.at[0], sem).wait()

    def wait4k(sem):
        pltpu.make_async_copy(src_h.at[pl.ds(0, 1024)],
                              kbuf.at[pl.ds(0, 1024)], sem).wait()

    for cst, sh, hrange in _HASH_LEVELS:
        ucst = jnp.uint32(cst)
        ush = jnp.uint32(sh)
        stripe = hrange // 16
        sbase = s * stripe

        for off in range(0, stripe, 1024):
            pltpu.async_copy(neg1, tkeys.at[pl.ds(sbase + off, 1024)], sga)
            pltpu.async_copy(zf, tvals.at[pl.ds(sbase + off, 1024)], sgb)
        for off in range(0, stripe, 1024):
            wait4k(sga)
            wait4k(sgb)
        plsc.subcore_barrier()

        def hash_row(buf, i, slot):
            for j in range(8):
                k16 = buf[pl.ds(i * 128 + j * 16, 16)]
                h = (k16.astype(jnp.uint32) * ucst) >> ush
                idxrow[slot, pl.ds(j * 16, 16)] = jnp.where(
                    k16 < 0, jnp.int32(_GARB), h.astype(jnp.int32))

        def bscat(p, carry):
            for slot, sem in ((0, saa), (1, sab)):
                i = 2 * p + slot

                @pl.when(p >= 1)
                def _guard():
                    wait512(sem)

                hash_row(kbuf, i, slot)
                pltpu.async_copy(kbuf.at[pl.ds(i * 128, 128)],
                                 tkeys.at[idxrow.at[slot]], sem)
            return carry

        lax.fori_loop(0, _BC // 2, bscat, 0)
        wait512(saa)
        wait512(sab)
        plsc.subcore_barrier()

        for slot, sem in ((0, sga), (1, sgb)):
            hash_row(kbuf, slot, slot)
            pltpu.async_copy(tkeys.at[idxrow.at[slot]], win.at[slot], sem)

        def bfin(p, carry):
            for slot, gsem, asem in ((0, sga, saa), (1, sgb, sab)):
                i = 2 * p + slot
                wait512(gsem)

                @pl.when(p >= 1)
                def _vguard():
                    wait512(asem)

                for j in range(8):
                    sl = pl.ds(i * 128 + j * 16, 16)
                    j16 = pl.ds(j * 16, 16)
                    k16 = kbuf[sl]
                    won = (win[slot, j16] == k16) & (k16 >= 0)
                    vst[slot, j16] = jnp.where(won, wbuf[sl], 0.0)
                    kbuf[sl] = jnp.where(won, -5, k16)
                    idxrow[2 + slot, j16] = idxrow[slot, j16]
                pltpu.async_copy(vst.at[slot],
                                 tvals.at[idxrow.at[2 + slot]], asem,
                                 add=True)

                @pl.when(p + 1 < _BC // 2)
                def _prefetch():
                    hash_row(kbuf, i + 2, slot)
                    pltpu.async_copy(tkeys.at[idxrow.at[slot]],
                                     win.at[slot], gsem)
            return carry

        lax.fori_loop(0, _BC // 2, bfin, 0)
        wait512(saa)
        wait512(sab)
        plsc.subcore_barrier()

        def qgather(i, slot):
            hash_row(qbuf, i, slot)
            pltpu.async_copy(tkeys.at[idxrow.at[slot]], qk.at[slot],
                             sga if slot == 0 else sgb)
            pltpu.async_copy(tvals.at[idxrow.at[slot]], qv.at[slot],
                             sga if slot == 0 else sgb)

        def qaccum(i, slot):
            for j in range(8):
                sl = pl.ds(i * 128 + j * 16, 16)
                j16 = pl.ds(j * 16, 16)
                found = qk[slot, j16] == qbuf[sl]
                acc[sl] = acc[sl] + jnp.where(found, qv[slot, j16], 0.0)

        qgather(0, 0)
        qgather(1, 1)

        def qprobe(p, carry):
            for slot, gsem in ((0, sga), (1, sgb)):
                i = 2 * p + slot
                wait512(gsem)
                wait512(gsem)
                qaccum(i, slot)

                @pl.when(2 * p + slot + 2 < _QC)
                def _prefetch():
                    qgather(i + 2, slot)
            return carry

        lax.fori_loop(0, _QC // 2, qprobe, 0)
        wait512(sga)
        wait512(sga)
        qaccum(_QC - 1, 0)
        plsc.subcore_barrier()

    pltpu.sync_copy(acc, out_h.at[pl.ds(qbase, _QT)])


@jax.jit
def _sc_join(src_p, dst_p, w_p):
    mesh = plsc.VectorSubcoreMesh(core_axis_name="c", subcore_axis_name="s")
    return pl.kernel(
        _join_body,
        out_type=jax.ShapeDtypeStruct((_EP,), jnp.float32),
        mesh=mesh,
        scratch_types=[
            pltpu.VMEM((_BT,), jnp.int32),
            pltpu.VMEM((_BT,), jnp.float32),
            pltpu.VMEM((_QT,), jnp.int32),
            pltpu.VMEM((_QT,), jnp.float32),
            pltpu.VMEM((4, 128), jnp.int32),
            pltpu.VMEM((2, 128), jnp.int32),
            pltpu.VMEM((2, 128), jnp.float32),
            pltpu.VMEM((2, 128), jnp.int32),
            pltpu.VMEM((2, 128), jnp.float32),
            pltpu.VMEM((256,), jnp.int32),
            pltpu.VMEM((1024,), jnp.int32),
            pltpu.VMEM((1024,), jnp.float32),
            pltpu.VMEM_SHARED((_TBL,), jnp.int32),
            pltpu.VMEM_SHARED((_TBL,), jnp.float32),
            pltpu.SemaphoreType.DMA,
            pltpu.SemaphoreType.DMA,
            pltpu.SemaphoreType.DMA,
            pltpu.SemaphoreType.DMA,
        ],
    )(src_p, dst_p, w_p)


def _epilogue_body(src_ref, dst_ref, inner_ref, aa_ref, bb_ref, wsum_ref,
                   out_ref):
    src = src_ref[...]
    dst = dst_ref[...]
    inner = inner_ref[...]
    denom = jnp.sqrt(aa_ref[...]) * jnp.sqrt(bb_ref[...]) + 1e-08
    keep = (inner / denom) >= _THR
    v = jnp.where(keep, wsum_ref[...], 0.0)
    out_ref[...] = jnp.where(src == dst, 2.0 * v, v)


def kernel(edge_index, edge_weight, features):
    E = edge_index.shape[1]
    src = edge_index[0]
    dst = edge_index[1]

    pad = _EP - E
    src_p = jnp.concatenate([src, jnp.full((pad,), _N, jnp.int32)])
    dst_p = jnp.concatenate([dst, jnp.zeros((pad,), jnp.int32)])
    w_p = jnp.concatenate([edge_weight, jnp.zeros((pad,), jnp.float32)])

    wsum = _sc_join(src_p, dst_p, w_p)[:E]

    fs = features[src]
    fd = features[dst]
    inner = jnp.sum(fs * fd, axis=-1)
    aa = jnp.sum(fs * fs, axis=-1)
    bb = jnp.sum(fd * fd, axis=-1)

    shape2d = (E // 128, 128)
    out = pl.pallas_call(
        _epilogue_body,
        out_shape=jax.ShapeDtypeStruct(shape2d, jnp.float32),
    )(
        src.reshape(shape2d),
        dst.reshape(shape2d),
        inner.reshape(shape2d),
        aa.reshape(shape2d),
        bb.reshape(shape2d),
        wsum.reshape(shape2d),
    )
    return out.reshape(E)
